# baseline (speedup 1.0000x reference)
import functools

import jax
import jax.numpy as jnp
from jax import lax
from jax.experimental import pallas as pl
from jax.experimental.pallas import tpu as pltpu
from jax.experimental.pallas import tpu_sc as plsc

N = 10000
E = 320000
D = 128
B = 256
NE = 16
H = 32

NC = 2
NS = 16
NW = NC * NS

CE = 128
CH = 80
PH = 2
CP = CH // PH
EPW = CH * CE
E_PAD = NW * EPW
N_PAD = 10240
RZ = N_PAD // NS
RO = 632
RO_LAST = N - RO * (NS - 1)

NB = 10
BN = N // NB
F = D + NE * H
NEP = 128


def _sc_agg_body(x_hbm, src_hbm, dst_hbm, out_hbm, agg_sh, src_v, dst_v,
                 rows0_v, sem0):
    c = lax.axis_index("c")
    s = lax.axis_index("s")
    w = s * NC + c

    def _zrow(r, carry):
        for k in range(CE // 16):
            rows0_v[r, pl.ds(k * 16, 16)] = jnp.zeros((16,), jnp.float32)
        return carry

    lax.fori_loop(0, CE, _zrow, 0)
    for z in range(RZ // CE):
        pltpu.sync_copy(rows0_v, agg_sh.at[pl.ds(s * RZ + z * CE, CE)])
    plsc.subcore_barrier()

    pltpu.sync_copy(src_hbm.at[w], src_v)
    pltpu.sync_copy(dst_hbm.at[w], dst_v)

    def _chunk(j, carry):
        pltpu.async_copy(x_hbm.at[src_v.at[j]], rows0_v, sem0).wait()
        pltpu.sync_copy(rows0_v, agg_sh.at[dst_v.at[j]], add=True)
        return carry

    lax.fori_loop(0, CH, _chunk, 0)
    plsc.subcore_barrier()

    @pl.when(s < NS - 1)
    def _():
        pltpu.sync_copy(agg_sh.at[pl.ds(s * RO, RO)],
                        out_hbm.at[c].at[pl.ds(s * RO, RO)])

    @pl.when(s == NS - 1)
    def _():
        pltpu.sync_copy(agg_sh.at[pl.ds((NS - 1) * RO, RO_LAST)],
                        out_hbm.at[c].at[pl.ds((NS - 1) * RO, RO_LAST)])


@functools.lru_cache(maxsize=1)
def _get_sc_agg():
    return pl.kernel(
        _sc_agg_body,
        out_type=jax.ShapeDtypeStruct((NC, N, D), jnp.float32),
        mesh=plsc.VectorSubcoreMesh(core_axis_name="c", subcore_axis_name="s",
                                    num_cores=NC, num_subcores=NS),
        scratch_types=[
            pltpu.VMEM_SHARED((N_PAD, D), jnp.float32),
            pltpu.VMEM((CH, CE), jnp.int32),
            pltpu.VMEM((CH, CE), jnp.int32),
            pltpu.VMEM((CE, D), jnp.float32),
            pltpu.SemaphoreType.DMA,
        ],
    )


def _tc_body(x_ref, agg_ref, batch_ref, wcat_ref, wg_ref, bg_ref, w2_ref,
             rw_ref, fo_ref, seg_scr):
    i = pl.program_id(0)
    node_in = x_ref[...] + agg_ref[0] + agg_ref[1]
    h = jnp.maximum(
        jnp.dot(node_in.astype(jnp.bfloat16),
                wcat_ref[...].astype(jnp.bfloat16),
                preferred_element_type=jnp.float32),
        0.0)
    bvec = batch_ref[0]
    onehot = (lax.broadcasted_iota(jnp.int32, (B, BN), 0)
              == bvec).astype(jnp.float32)
    contrib = jnp.dot(onehot, h, preferred_element_type=jnp.float32,
                precision=lax.Precision.HIGHEST)

    @pl.when(i == 0)
    def _():
        seg_scr[...] = contrib

    @pl.when(i > 0)
    def _():
        seg_scr[...] += contrib

    @pl.when(i == NB - 1)
    def _():
        seg = seg_scr[...]
        graph_emb = seg[:, :D]
        ge = seg[:, D:]
        logits = (jnp.dot(graph_emb.astype(jnp.bfloat16),
                          wg_ref[...].astype(jnp.bfloat16),
                          preferred_element_type=jnp.float32) + bg_ref[...])
        mx = jnp.max(logits, axis=1, keepdims=True)
        ex = jnp.exp(logits - mx)
        rw = ex / jnp.sum(ex, axis=1, keepdims=True)
        rw_ref[...] = rw

        eo = jnp.dot(ge.astype(jnp.bfloat16),
                     w2_ref[...].astype(jnp.bfloat16),
                     preferred_element_type=jnp.float32)

        col = lax.broadcasted_iota(jnp.int32, (B, NEP), 1)
        big = jnp.int32(NEP)
        w1 = jnp.max(rw, axis=1, keepdims=True)
        i1 = jnp.min(jnp.where(rw == w1, col, big), axis=1, keepdims=True)
        rw2 = jnp.where(col == i1, jnp.float32(-1.0), rw)
        w2 = jnp.max(rw2, axis=1, keepdims=True)
        i2 = jnp.min(jnp.where(rw2 == w2, col, big), axis=1, keepdims=True)
        s1 = jnp.sum(jnp.where(col == i1, eo, 0.0), axis=1, keepdims=True)
        s2 = jnp.sum(jnp.where(col == i2, eo, 0.0), axis=1, keepdims=True)
        t = jnp.exp(w2 - w1)
        denom = 1.0 + t
        final = (s1 + s2 * t) / denom
        fo_ref[...] = jnp.broadcast_to(final, (B, NEP))


def _tc_head(x, agg, batch3, wcat, wg_pad, bg_pad, w2_pad):
    return pl.pallas_call(
        _tc_body,
        grid=(NB,),
        in_specs=[
            pl.BlockSpec((BN, D), lambda i: (i, 0)),
            pl.BlockSpec((NC, BN, D), lambda i: (0, i, 0)),
            pl.BlockSpec((1, 1, BN), lambda i: (i, 0, 0)),
            pl.BlockSpec((D, F), lambda i: (0, 0)),
            pl.BlockSpec((D, NEP), lambda i: (0, 0)),
            pl.BlockSpec((1, NEP), lambda i: (0, 0)),
            pl.BlockSpec((NE * H, NEP), lambda i: (0, 0)),
        ],
        out_specs=[
            pl.BlockSpec((B, NEP), lambda i: (0, 0)),
            pl.BlockSpec((B, NEP), lambda i: (0, 0)),
        ],
        out_shape=[
            jax.ShapeDtypeStruct((B, NEP), jnp.float32),
            jax.ShapeDtypeStruct((B, NEP), jnp.float32),
        ],
        scratch_shapes=[pltpu.VMEM((B, F), jnp.float32)],
        compiler_params=pltpu.CompilerParams(
            dimension_semantics=("arbitrary",)),
    )(x, agg, batch3, wcat, wg_pad, bg_pad, w2_pad)


def kernel(x, edge_src, edge_dst, batch, W_emb, Wg, bg, W1s, W2s):
    pad = E_PAD - E
    src_p = jnp.concatenate(
        [edge_src, jnp.zeros((pad,), jnp.int32)]).reshape(NW, CH, CE)
    dst_p = jnp.concatenate(
        [edge_dst, jnp.full((pad,), N, jnp.int32)]).reshape(NW, CH, CE)

    agg = _get_sc_agg()(x, src_p, dst_p)

    batch3 = batch.reshape(NB, 1, BN)
    wcat = jnp.concatenate(
        [W_emb, W1s.transpose(1, 0, 2).reshape(D, NE * H)], axis=1)
    wg_pad = jnp.pad(Wg, ((0, 0), (0, NEP - NE)))
    bg_pad = jnp.pad(bg, (0, NEP - NE),
                     constant_values=-1e30).reshape(1, NEP)
    w2_blk = (W2s[:, :, 0][:, :, None]
              * jnp.eye(NE, dtype=jnp.float32)[:, None, :]).reshape(
                  NE * H, NE)
    w2_pad = jnp.pad(w2_blk, ((0, 0), (0, NEP - NE)))

    rw_pad, fo_pad = _tc_head(x, agg, batch3, wcat, wg_pad, bg_pad, w2_pad)
    return fo_pad[:, :1], rw_pad[:, :NE]


# speedup vs baseline: 1.4908x; 1.4908x over previous
import functools

import jax
import jax.numpy as jnp
from jax import lax
from jax.experimental import pallas as pl
from jax.experimental.pallas import tpu as pltpu
from jax.experimental.pallas import tpu_sc as plsc

N = 10000
E = 320000
D = 128
B = 256
NE = 16
H = 32

NC = 2
NS = 16
NW = NC * NS

CE = 128
CH = 79
EPW = CH * CE
E_PAD = NW * EPW
N_PAD = 10240
RZ = N_PAD // NS
RO = 632
RO_LAST = N - RO * (NS - 1)

NB = 10
BN = N // NB
F = D + NE * H
NEP = 128


def _sc_agg_body(x_hbm, src_hbm, dst_hbm, out_hbm, agg_sh, src_v, dst_v,
                 rows0_v, sem0):
    c = lax.axis_index("c")
    s = lax.axis_index("s")
    w = s * NC + c

    def _zrow(r, carry):
        for k in range(CE // 16):
            rows0_v[r, pl.ds(k * 16, 16)] = jnp.zeros((16,), jnp.float32)
        return carry

    lax.fori_loop(0, CE, _zrow, 0)
    for z in range(RZ // CE):
        pltpu.sync_copy(rows0_v, agg_sh.at[pl.ds(s * RZ + z * CE, CE)])

    pltpu.sync_copy(src_hbm.at[w], src_v)
    pltpu.sync_copy(dst_hbm.at[w], dst_v)
    plsc.subcore_barrier()

    def _chunk(j, carry):
        pltpu.async_copy(x_hbm.at[src_v.at[j]], rows0_v, sem0).wait()
        pltpu.sync_copy(rows0_v, agg_sh.at[dst_v.at[j]], add=True)
        return carry

    lax.fori_loop(0, CH, _chunk, 0)
    plsc.subcore_barrier()

    @pl.when(s < NS - 1)
    def _():
        pltpu.sync_copy(agg_sh.at[pl.ds(s * RO, RO)],
                        out_hbm.at[c].at[pl.ds(s * RO, RO)])

    @pl.when(s == NS - 1)
    def _():
        pltpu.sync_copy(agg_sh.at[pl.ds((NS - 1) * RO, RO_LAST)],
                        out_hbm.at[c].at[pl.ds((NS - 1) * RO, RO_LAST)])


@functools.lru_cache(maxsize=1)
def _get_sc_agg():
    return pl.kernel(
        _sc_agg_body,
        out_type=jax.ShapeDtypeStruct((NC, N, D), jnp.float32),
        mesh=plsc.VectorSubcoreMesh(core_axis_name="c", subcore_axis_name="s",
                                    num_cores=NC, num_subcores=NS),
        scratch_types=[
            pltpu.VMEM_SHARED((N_PAD, D), jnp.float32),
            pltpu.VMEM((CH, CE), jnp.int32),
            pltpu.VMEM((CH, CE), jnp.int32),
            pltpu.VMEM((CE, D), jnp.float32),
            pltpu.SemaphoreType.DMA,
        ],
    )


def _tc_body(x_ref, agg_ref, batch_ref, wcat_ref, wg_ref, bg_ref, w2_ref,
             rw_ref, fo_ref, seg_scr):
    i = pl.program_id(0)
    node_in = x_ref[...] + agg_ref[0] + agg_ref[1]
    h = jnp.maximum(
        jnp.dot(node_in.astype(jnp.bfloat16),
                wcat_ref[...].astype(jnp.bfloat16),
                preferred_element_type=jnp.float32),
        0.0)
    bvec = batch_ref[0]
    onehot = (lax.broadcasted_iota(jnp.int32, (B, BN), 0)
              == bvec).astype(jnp.float32)
    contrib = jnp.dot(onehot, h, preferred_element_type=jnp.float32,
                precision=lax.Precision.HIGHEST)

    @pl.when(i == 0)
    def _():
        seg_scr[...] = contrib

    @pl.when(i > 0)
    def _():
        seg_scr[...] += contrib

    @pl.when(i == NB - 1)
    def _():
        seg = seg_scr[...]
        graph_emb = seg[:, :D]
        ge = seg[:, D:]
        logits = (jnp.dot(graph_emb.astype(jnp.bfloat16),
                          wg_ref[...].astype(jnp.bfloat16),
                          preferred_element_type=jnp.float32) + bg_ref[...])
        mx = jnp.max(logits, axis=1, keepdims=True)
        ex = jnp.exp(logits - mx)
        rw = ex / jnp.sum(ex, axis=1, keepdims=True)
        rw_ref[...] = rw

        eo = jnp.dot(ge.astype(jnp.bfloat16),
                     w2_ref[...].astype(jnp.bfloat16),
                     preferred_element_type=jnp.float32)

        col = lax.broadcasted_iota(jnp.int32, (B, NEP), 1)
        big = jnp.int32(NEP)
        w1 = jnp.max(rw, axis=1, keepdims=True)
        i1 = jnp.min(jnp.where(rw == w1, col, big), axis=1, keepdims=True)
        rw2 = jnp.where(col == i1, jnp.float32(-1.0), rw)
        w2 = jnp.max(rw2, axis=1, keepdims=True)
        i2 = jnp.min(jnp.where(rw2 == w2, col, big), axis=1, keepdims=True)
        s1 = jnp.sum(jnp.where(col == i1, eo, 0.0), axis=1, keepdims=True)
        s2 = jnp.sum(jnp.where(col == i2, eo, 0.0), axis=1, keepdims=True)
        t = jnp.exp(w2 - w1)
        denom = 1.0 + t
        final = (s1 + s2 * t) / denom
        fo_ref[...] = jnp.broadcast_to(final, (B, NEP))


def _tc_head(x, agg, batch3, wcat, wg_pad, bg_pad, w2_pad):
    return pl.pallas_call(
        _tc_body,
        grid=(NB,),
        in_specs=[
            pl.BlockSpec((BN, D), lambda i: (i, 0)),
            pl.BlockSpec((NC, BN, D), lambda i: (0, i, 0)),
            pl.BlockSpec((1, 1, BN), lambda i: (i, 0, 0)),
            pl.BlockSpec((D, F), lambda i: (0, 0)),
            pl.BlockSpec((D, NEP), lambda i: (0, 0)),
            pl.BlockSpec((1, NEP), lambda i: (0, 0)),
            pl.BlockSpec((NE * H, NEP), lambda i: (0, 0)),
        ],
        out_specs=[
            pl.BlockSpec((B, NEP), lambda i: (0, 0)),
            pl.BlockSpec((B, NEP), lambda i: (0, 0)),
        ],
        out_shape=[
            jax.ShapeDtypeStruct((B, NEP), jnp.float32),
            jax.ShapeDtypeStruct((B, NEP), jnp.float32),
        ],
        scratch_shapes=[pltpu.VMEM((B, F), jnp.float32)],
        compiler_params=pltpu.CompilerParams(
            dimension_semantics=("arbitrary",)),
    )(x, agg, batch3, wcat, wg_pad, bg_pad, w2_pad)


def kernel(x, edge_src, edge_dst, batch, W_emb, Wg, bg, W1s, W2s):
    pad = E_PAD - E
    src_p = jnp.concatenate(
        [edge_src, jnp.zeros((pad,), jnp.int32)]).reshape(NW, CH, CE)
    dst_p = jnp.concatenate(
        [edge_dst, jnp.full((pad,), N, jnp.int32)]).reshape(NW, CH, CE)

    agg = _get_sc_agg()(x, src_p, dst_p)

    batch3 = batch.reshape(NB, 1, BN)
    wcat = jnp.concatenate(
        [W_emb, W1s.transpose(1, 0, 2).reshape(D, NE * H)], axis=1)
    wg_pad = jnp.pad(Wg, ((0, 0), (0, NEP - NE)))
    bg_pad = jnp.pad(bg, (0, NEP - NE),
                     constant_values=-1e30).reshape(1, NEP)
    w2_blk = (W2s[:, :, 0][:, :, None]
              * jnp.eye(NE, dtype=jnp.float32)[:, None, :]).reshape(
                  NE * H, NE)
    w2_pad = jnp.pad(w2_blk, ((0, 0), (0, NEP - NE)))

    rw_pad, fo_pad = _tc_head(x, agg, batch3, wcat, wg_pad, bg_pad, w2_pad)
    return fo_pad[:, :1], rw_pad[:, :NE]
